# Initial kernel scaffold; baseline (speedup 1.0000x reference)
import jax
import jax.numpy as jnp
from jax.experimental import pallas as pl


def kernel(turn, turns_since_last_move, action_type_mask, move_mask, max_move_mask, switch_mask, flag_mask, target_mask, n, total_pokemon, faint_counter, W1, b1, W2, b2):
    raise NotImplementedError("write your pallas kernel here")


# speedup vs baseline: 5.5551x; 5.5551x over previous
import functools

import jax
import jax.numpy as jnp
from jax.experimental import pallas as pl
from jax.experimental.pallas import tpu as pltpu

_B = 16384
_D = 128
_FEAT_PAD = 96


def _encoder_body(turn_ref, tslm_ref, action_ref, move_ref, maxmove_ref,
                  switch_ref, flag_ref, n_ref, tp_ref, fc_ref,
                  w1_ref, b1_ref, w2_ref, b2_ref, out_ref):
    bm = out_ref.shape[0]
    turn = jnp.clip(turn_ref[:, 0:1], 0, 200)
    tslm = jnp.clip(tslm_ref[:, 0:1], 0, 50)
    turn_f = turn.astype(jnp.float32)
    tslm_f = tslm.astype(jnp.float32)
    i1 = jnp.floor(jnp.sqrt(turn_f)).astype(jnp.int32)
    i2 = jnp.floor(jnp.sqrt(tslm_f)).astype(jnp.int32)

    def onehot(idx, k):
        io = jax.lax.broadcasted_iota(jnp.int32, (bm, k), 1)
        return (idx == io).astype(jnp.float32)

    feat = jnp.concatenate([
        onehot(i1, 15),
        onehot(i2, 8),
        turn_f * (1.0 / 200.0),
        tslm_f * (1.0 / 50.0),
        action_ref[...],
        move_ref[...],
        switch_ref[...],
        flag_ref[...],
        onehot(n_ref[:, 0:1], 7),
        onehot(n_ref[:, 1:2], 7),
        onehot(tp_ref[:, 0:1], 7),
        onehot(tp_ref[:, 1:2], 7),
        onehot(fc_ref[:, 0:1], 7),
        onehot(fc_ref[:, 1:2], 7),
        maxmove_ref[...],
        jnp.zeros((bm, _FEAT_PAD - 89), jnp.float32),
    ], axis=1)
    h = jnp.dot(feat, w1_ref[...], preferred_element_type=jnp.float32)
    h = jnp.maximum(h + b1_ref[...], 0.0)
    out_ref[...] = jnp.dot(h, w2_ref[...],
                           preferred_element_type=jnp.float32) + b2_ref[...]


@functools.partial(jax.jit, static_argnames=("bm",))
def _run(turn, tslm, action, move, maxmove, switch, flag, n, tp, fc,
         w1p, b1, w2, b2, bm=1024):
    grid = (_B // bm,)
    row = lambda w: pl.BlockSpec((bm, w), lambda i: (i, 0))
    full = lambda a, b: pl.BlockSpec((a, b), lambda i: (0, 0))
    return pl.pallas_call(
        _encoder_body,
        grid=grid,
        in_specs=[
            row(1), row(1), row(3), row(4), row(4), row(6), row(5),
            row(2), row(2), row(2),
            full(_FEAT_PAD, _D), full(1, _D), full(_D, _D), full(1, _D),
        ],
        out_specs=row(_D),
        out_shape=jax.ShapeDtypeStruct((_B, _D), jnp.float32),
    )(turn, tslm, action, move, maxmove, switch, flag, n, tp, fc,
      w1p, b1, w2, b2)


def kernel(turn, turns_since_last_move, action_type_mask, move_mask,
           max_move_mask, switch_mask, flag_mask, target_mask, n,
           total_pokemon, faint_counter, W1, b1, W2, b2):
    del target_mask
    w1p = jnp.zeros((_FEAT_PAD, _D), jnp.float32).at[:89].set(W1)
    return _run(
        turn.reshape(_B, 1).astype(jnp.int32),
        turns_since_last_move.reshape(_B, 1).astype(jnp.int32),
        action_type_mask, move_mask, max_move_mask, switch_mask, flag_mask,
        n.astype(jnp.int32), total_pokemon.astype(jnp.int32),
        faint_counter.astype(jnp.int32),
        w1p, b1.reshape(1, _D), W2, b2.reshape(1, _D))
